# baseline (speedup 1.0000x reference)
import functools

import jax
import jax.numpy as jnp
from jax import lax
from jax.experimental import pallas as pl
from jax.experimental.pallas import tpu as pltpu
from jax.experimental.pallas import tpu_sc as plsc

N = 10000
E = 160000
NS = 16
NC = 2
W = 128
EDGE_B = 80
NBLK = E // NS // EDGE_B
HALF_B = 40
HALF_NBLK = E // (NC * NS) // HALF_B
NBUF = 4
TCR = 1000


def _sc_mesh():
    return plsc.VectorSubcoreMesh(core_axis_name="c", subcore_axis_name="s")


def _scatter_pass(nblk, b_sz, ebase, dst_hbm, onesv, didx, acc, isems, ssems):
    def issue_idx(b, j):
        off = pl.multiple_of(ebase + b * b_sz, 8)
        pltpu.async_copy(dst_hbm.at[pl.ds(off, b_sz)], didx[j], isems[j])

    for j in range(NBUF):
        @pl.when(j < nblk)
        def _(j=j):
            issue_idx(j, j)

    @pl.loop(0, nblk, step=NBUF)
    def _(g):
        for j in range(NBUF):
            b = g + j

            @pl.when(b < nblk)
            def _(b=b, j=j):
                pltpu.make_async_copy(dst_hbm.at[pl.ds(0, b_sz)], didx[j],
                                      isems[j]).wait()
                pltpu.async_copy(onesv, acc.at[didx[j]], ssems[j], add=True)

        for j in range(NBUF):
            b = g + j

            @pl.when(b + NBUF < nblk)
            def _(b=b, j=j):
                pltpu.make_async_copy(onesv, acc.at[didx[j]], ssems[j]).wait()
                issue_idx(b + NBUF, j)

    for j in range(NBUF):
        pltpu.make_async_copy(onesv, acc.at[didx[j]], ssems[j]).wait()


def _gather_scatter_pass(nblk, b_sz, ebase, src_hbm, dst_hbm, y_hbm,
                         sidx, didx, rows, acc, isems, gsems, ssems,
                         idx_off=None, idxbuf=None):
    def issue_idx(b, j):
        off = pl.multiple_of(ebase + b * b_sz, 8)
        pltpu.async_copy(src_hbm.at[pl.ds(off, b_sz)], sidx[j], isems[j])
        pltpu.async_copy(dst_hbm.at[pl.ds(off, b_sz)], didx[j], isems[j])

    def wait_idx(j):
        pltpu.make_async_copy(src_hbm.at[pl.ds(0, b_sz)], sidx[j], isems[j]).wait()
        pltpu.make_async_copy(dst_hbm.at[pl.ds(0, b_sz)], didx[j], isems[j]).wait()

    def gather_idx_ref(j):
        return idxbuf[j] if idx_off is not None else sidx[j]

    def issue_gather(j):
        if idx_off is not None:
            for k in range(b_sz // 16):
                sl = pl.ds(k * 16, 16)
                idxbuf[j][sl] = sidx[j][sl] * 2 + idx_off
        pltpu.async_copy(y_hbm.at[gather_idx_ref(j)], rows[j], gsems[j])

    for j in range(NBUF):
        @pl.when(j < nblk)
        def _(j=j):
            issue_idx(j, j)

    @pl.loop(0, nblk, step=NBUF)
    def _(g):
        for j in range(NBUF):
            b = g + j

            @pl.when(b < nblk)
            def _(b=b, j=j):
                wait_idx(j)
                issue_gather(j)

        for j in range(NBUF):
            b = g + j

            @pl.when(b < nblk)
            def _(b=b, j=j):
                pltpu.make_async_copy(y_hbm.at[gather_idx_ref(j)], rows[j],
                                      gsems[j]).wait()
                pltpu.async_copy(rows[j], acc.at[didx[j]], ssems[j], add=True)

        for j in range(NBUF):
            b = g + j

            @pl.when(b + NBUF < nblk)
            def _(b=b, j=j):
                pltpu.make_async_copy(rows[j], acc.at[didx[j]], ssems[j]).wait()
                issue_idx(b + NBUF, j)

    for j in range(NBUF):
        pltpu.make_async_copy(rows[j], acc.at[didx[j]], ssems[j]).wait()


_WB = 624
_WB_TAIL = N - NS * _WB


def _copy_sliced(s, src, dst):
    off = s * _WB
    pltpu.sync_copy(src.at[pl.ds(off, _WB)], dst.at[pl.ds(off, _WB)])

    @pl.when(s == NS - 1)
    def _():
        pltpu.sync_copy(src.at[pl.ds(NS * _WB, _WB_TAIL)],
                        dst.at[pl.ds(NS * _WB, _WB_TAIL)])


def _zero_acc(s, zeros_hbm, acc):
    _copy_sliced(s, zeros_hbm, acc)


def _core_writeback(c, s, acc, out0_hbm, out1_hbm):
    @pl.when(c == 0)
    def _():
        _copy_sliced(s, acc, out0_hbm)

    @pl.when(c == 1)
    def _():
        _copy_sliced(s, acc, out1_hbm)


def _make_deg_kernel():
    @functools.partial(
        pl.kernel,
        out_type=[jax.ShapeDtypeStruct((N, W), jnp.float32),
                  jax.ShapeDtypeStruct((N, W), jnp.float32)],
        mesh=_sc_mesh(),
        scratch_types=[pltpu.VMEM((HALF_B,), jnp.int32)] * NBUF
          + [pltpu.VMEM((HALF_B, W), jnp.float32),
             pltpu.VMEM_SHARED((N, W), jnp.float32)]
          + [pltpu.SemaphoreType.DMA] * (2 * NBUF),
    )
    def deg_k(dst_hbm, ones_hbm, zeros_hbm, out0_hbm, out1_hbm, *scr):
        didx = list(scr[0:NBUF])
        onesv, acc = scr[NBUF], scr[NBUF + 1]
        sems = list(scr[NBUF + 2:])
        isems, ssems = sems[:NBUF], sems[NBUF:]

        c = lax.axis_index("c")
        s = lax.axis_index("s")

        _zero_acc(s, zeros_hbm, acc)

        pltpu.sync_copy(ones_hbm, onesv)
        plsc.subcore_barrier()
        ebase = (c * NS + s) * HALF_NBLK * HALF_B
        _scatter_pass(HALF_NBLK, HALF_B, ebase, dst_hbm, onesv, didx, acc,
                      isems, ssems)
        plsc.subcore_barrier()
        _core_writeback(c, s, acc, out0_hbm, out1_hbm)

    return deg_k


def _make_layer1_kernel():
    @functools.partial(
        pl.kernel,
        out_type=[jax.ShapeDtypeStruct((N, W), jnp.float32),
                  jax.ShapeDtypeStruct((N, W), jnp.float32),
                  jax.ShapeDtypeStruct((N, W), jnp.float32),
                  jax.ShapeDtypeStruct((N, W), jnp.float32)],
        mesh=_sc_mesh(),
        scratch_types=[pltpu.VMEM((EDGE_B,), jnp.int32)] * (3 * NBUF)
          + [pltpu.VMEM((HALF_B,), jnp.int32)] * (2 * NBUF)
          + [pltpu.VMEM((EDGE_B, W), jnp.float32)] * NBUF
          + [pltpu.VMEM_SHARED((N, W), jnp.float32)]
          + [pltpu.SemaphoreType.DMA] * (3 * NBUF),
    )
    def l1_k(x2d_hbm, u_hbm, src_hbm, dst_hbm,
             zeros_hbm, sa_hbm, sb_hbm, up0_hbm, up1_hbm, *scr):
        sidx = list(scr[0:NBUF])
        didx = list(scr[NBUF:2 * NBUF])
        idxbuf = list(scr[2 * NBUF:3 * NBUF])
        sidxh = list(scr[3 * NBUF:4 * NBUF])
        didxh = list(scr[4 * NBUF:5 * NBUF])
        rows = list(scr[5 * NBUF:6 * NBUF])
        acc = scr[6 * NBUF]
        sems = list(scr[6 * NBUF + 1:])
        isems, gsems, ssems = sems[:NBUF], sems[NBUF:2 * NBUF], sems[2 * NBUF:]

        c = lax.axis_index("c")
        s = lax.axis_index("s")

        _zero_acc(s, zeros_hbm, acc)

        plsc.subcore_barrier()
        _gather_scatter_pass(NBLK, EDGE_B, s * NBLK * EDGE_B, src_hbm, dst_hbm,
                             x2d_hbm, sidx, didx, rows, acc,
                             isems, gsems, ssems, idx_off=c, idxbuf=idxbuf)
        plsc.subcore_barrier()
        _core_writeback(c, s, acc, sa_hbm, sb_hbm)
        plsc.subcore_barrier()

        _zero_acc(s, zeros_hbm, acc)

        plsc.subcore_barrier()
        rows40 = [r.at[pl.ds(0, HALF_B)] for r in rows]
        _gather_scatter_pass(HALF_NBLK, HALF_B,
                             (c * NS + s) * HALF_NBLK * HALF_B,
                             src_hbm, dst_hbm, u_hbm, sidxh, didxh, rows40,
                             acc, isems, gsems, ssems)
        plsc.subcore_barrier()
        _core_writeback(c, s, acc, up0_hbm, up1_hbm)

    return l1_k


def _make_layer2_kernel():
    @functools.partial(
        pl.kernel,
        out_type=[jax.ShapeDtypeStruct((N, W), jnp.float32),
                  jax.ShapeDtypeStruct((N, W), jnp.float32)],
        mesh=_sc_mesh(),
        scratch_types=[pltpu.VMEM((HALF_B,), jnp.int32)] * (2 * NBUF)
          + [pltpu.VMEM((HALF_B, W), jnp.float32)] * NBUF
          + [pltpu.VMEM_SHARED((N, W), jnp.float32)]
          + [pltpu.SemaphoreType.DMA] * (3 * NBUF),
    )
    def l2_k(x1_hbm, src_hbm, dst_hbm, zeros_hbm, p0_hbm, p1_hbm, *scr):
        sidx = list(scr[0:NBUF])
        didx = list(scr[NBUF:2 * NBUF])
        rows = list(scr[2 * NBUF:3 * NBUF])
        acc = scr[3 * NBUF]
        sems = list(scr[3 * NBUF + 1:])
        isems, gsems, ssems = sems[:NBUF], sems[NBUF:2 * NBUF], sems[2 * NBUF:]

        c = lax.axis_index("c")
        s = lax.axis_index("s")

        _zero_acc(s, zeros_hbm, acc)

        plsc.subcore_barrier()
        _gather_scatter_pass(HALF_NBLK, HALF_B,
                             (c * NS + s) * HALF_NBLK * HALF_B,
                             src_hbm, dst_hbm, x1_hbm, sidx, didx, rows,
                             acc, isems, gsems, ssems)
        plsc.subcore_barrier()
        _core_writeback(c, s, acc, p0_hbm, p1_hbm)

    return l2_k


def _full(shape):
    return pl.BlockSpec(shape, lambda i: tuple(0 for _ in shape))


def _rows(width):
    return pl.BlockSpec((TCR, width), lambda i: (i, 0))


def _tc1(x, gcn_w, w2, deg0, deg1):
    def body(x_ref, d0_ref, d1_ref, w_ref, w2_ref, u_ref, h_ref, xw_ref):
        deg = d0_ref[:, :1] + d1_ref[:, :1]
        dinv = lax.rsqrt(deg + 1.0)
        xv = x_ref[...]
        h = jnp.dot(xv, w_ref[...], preferred_element_type=jnp.float32)
        h_ref[...] = h
        u_ref[...] = dinv * h
        xw_ref[...] = jnp.dot(xv, w2_ref[...], preferred_element_type=jnp.float32)

    return pl.pallas_call(
        body,
        grid=(N // TCR,),
        in_specs=[_rows(256), _rows(W), _rows(W),
                  _full((256, 128)), _full((256, 256))],
        out_specs=[_rows(128), _rows(128), _rows(256)],
        out_shape=[jax.ShapeDtypeStruct((N, 128), jnp.float32),
                   jax.ShapeDtypeStruct((N, 128), jnp.float32),
                   jax.ShapeDtypeStruct((N, 256), jnp.float32)],
    )(x, deg0, deg1, gcn_w, w2)


def _tc2(sa, sb, up0, up1, h, xw, deg0, deg1, w3, gcn_b, sage_b, graph_brel):
    def body(sa_ref, sb_ref, u0_ref, u1_ref, h_ref, xw_ref, d0_ref, d1_ref,
             w3_ref, gb_ref, sb2_ref, rb_ref, x1_ref, x2_ref, x3_ref):
        deg = d0_ref[:, :1] + d1_ref[:, :1]
        dinv_sl = lax.rsqrt(deg + 1.0)
        inv_sl = 1.0 / (deg + 1.0)
        sinv = 1.0 / jnp.maximum(deg, 1.0)
        nsum = jnp.concatenate([sa_ref[...], sb_ref[...]], axis=1)
        gs = u0_ref[...] + u1_ref[...]
        gcn = dinv_sl * gs + h_ref[...] * inv_sl + gb_ref[...]
        x1_ref[...] = jnp.maximum(gcn, 0.0)
        nw = jnp.dot(nsum, w3_ref[...], preferred_element_type=jnp.float32)
        xwv = xw_ref[...]
        x2_ref[...] = jnp.maximum(sinv * nw[:, :128] + xwv[:, :128] + sb2_ref[...], 0.0)
        x3_ref[...] = jnp.maximum(nw[:, 128:] + xwv[:, 128:] + rb_ref[...], 0.0)

    return pl.pallas_call(
        body,
        grid=(N // TCR,),
        in_specs=[_rows(W), _rows(W), _rows(W), _rows(W),
                  _rows(128), _rows(256), _rows(W), _rows(W),
                  _full((256, 256)),
                  _full((1, 128)), _full((1, 128)), _full((1, 128))],
        out_specs=[_rows(128), _rows(128), _rows(128)],
        out_shape=[jax.ShapeDtypeStruct((N, 128), jnp.float32),
                   jax.ShapeDtypeStruct((N, 128), jnp.float32),
                   jax.ShapeDtypeStruct((N, 128), jnp.float32)],
    )(sa, sb, up0, up1, h, xw, deg0, deg1, w3,
      gcn_b.reshape(1, 128), sage_b.reshape(1, 128), graph_brel.reshape(1, 128))


def _tc3(x1, n2p0, n2p1, x2, x3, gin_w, gin_b, out_w, out_b):
    def body(x1_ref, p0_ref, p1_ref, x2_ref, x3_ref,
             gw_ref, gb_ref, ow_ref, ob_ref, out_ref):
        z = x1_ref[...] + p0_ref[...] + p1_ref[...]
        x4 = jnp.maximum(
            jnp.dot(z, gw_ref[...], preferred_element_type=jnp.float32)
            + gb_ref[...], 0.0)
        ow = ow_ref[...]
        o = (jnp.dot(x2_ref[...], ow[:128], preferred_element_type=jnp.float32)
             + jnp.dot(x3_ref[...], ow[128:256], preferred_element_type=jnp.float32)
             + jnp.dot(x4, ow[256:], preferred_element_type=jnp.float32)
             + ob_ref[...])
        out_ref[...] = jax.nn.sigmoid(o)

    return pl.pallas_call(
        body,
        grid=(N // TCR,),
        in_specs=[_rows(128), _rows(W), _rows(W), _rows(128), _rows(128),
                  _full((128, 128)), _full((1, 128)),
                  _full((384, 256)), _full((1, 256))],
        out_specs=_rows(256),
        out_shape=jax.ShapeDtypeStruct((N, 256), jnp.float32),
    )(x1, n2p0, n2p1, x2, x3,
      gin_w, gin_b.reshape(1, 128), out_w, out_b.reshape(1, 256))


def kernel(x, edge_index, params):
    p1, p2, po = params["l1"], params["l2"], params["out"]
    src = edge_index[0]
    dst = edge_index[1]

    ones128 = jnp.ones((HALF_B, W), jnp.float32)
    zeros128 = jnp.zeros((N, W), jnp.float32)
    deg0, deg1 = _make_deg_kernel()(dst, ones128, zeros128)

    w2 = jnp.concatenate([p1["sage_wr"], p1["graph_wroot"]], axis=1)
    u, h, xw = _tc1(x, p1["gcn_w"], w2, deg0, deg1)

    x2d = x.reshape(2 * N, 128)
    sa, sb, up0, up1 = _make_layer1_kernel()(x2d, u, src, dst, zeros128)

    w3 = jnp.concatenate([p1["sage_wl"], p1["graph_wrel"]], axis=1)
    x1, x2, x3 = _tc2(sa, sb, up0, up1, h, xw, deg0, deg1, w3,
                      p1["gcn_b"], p1["sage_b"], p1["graph_brel"])

    n2p0, n2p1 = _make_layer2_kernel()(x1, src, dst, zeros128)

    return _tc3(x1, n2p0, n2p1, x2, x3,
                p2["gin_w"], p2["gin_b"], po["w"], po["b"])


# speedup vs baseline: 1.1358x; 1.1358x over previous
import functools

import jax
import jax.numpy as jnp
from jax import lax
from jax.experimental import pallas as pl
from jax.experimental.pallas import tpu as pltpu
from jax.experimental.pallas import tpu_sc as plsc

N = 10000
E = 160000
NS = 16
NC = 2
W = 128
EDGE_B = 80
NBLK = E // NS // EDGE_B
HALF_B = 40
HALF_NBLK = E // (NC * NS) // HALF_B
NBUF = 4
TCR = 1000


def _sc_mesh():
    return plsc.VectorSubcoreMesh(core_axis_name="c", subcore_axis_name="s")


IDXN = 2 * NBUF


def _scatter_pass(nblk, b_sz, ebase, dst_hbm, onesv, didx, acc, isems, ssems):
    def issue_idx(b, i):
        off = pl.multiple_of(ebase + b * b_sz, 8)
        pltpu.async_copy(dst_hbm.at[pl.ds(off, b_sz)], didx[i], isems[i])

    def wait_idx(i):
        pltpu.make_async_copy(dst_hbm.at[pl.ds(0, b_sz)], didx[i],
                              isems[i]).wait()

    for i in range(IDXN):
        @pl.when(i < nblk)
        def _(i=i):
            issue_idx(i, i)

    @pl.loop(0, nblk, step=IDXN)
    def _(g):
        for half in range(2):
            for j in range(NBUF):
                b = g + NBUF * half + j
                i = NBUF * half + j

                @pl.when(b < nblk)
                def _(b=b, i=i, j=j):
                    wait_idx(i)
                    pltpu.async_copy(onesv, acc.at[didx[i]], ssems[j],
                                     add=True)

            for j in range(NBUF):
                b = g + NBUF * half + j
                i = NBUF * half + j

                @pl.when(b < nblk)
                def _(b=b, i=i, j=j):
                    pltpu.make_async_copy(onesv, acc.at[didx[i]],
                                          ssems[j]).wait()

                    @pl.when(b + IDXN < nblk)
                    def _():
                        issue_idx(b + IDXN, i)


def _gather_scatter_pass(nblk, b_sz, ebase, src_hbm, dst_hbm, y_hbm,
                         sidx, didx, rows, acc, isems, gsems, ssems,
                         idx_off=None, idxbuf=None):
    def issue_idx(b, i):
        off = pl.multiple_of(ebase + b * b_sz, 8)
        pltpu.async_copy(src_hbm.at[pl.ds(off, b_sz)], sidx[i], isems[i])
        pltpu.async_copy(dst_hbm.at[pl.ds(off, b_sz)], didx[i], isems[i])

    def wait_idx(i):
        pltpu.make_async_copy(src_hbm.at[pl.ds(0, b_sz)], sidx[i], isems[i]).wait()
        pltpu.make_async_copy(dst_hbm.at[pl.ds(0, b_sz)], didx[i], isems[i]).wait()

    def gather_idx_ref(i, j):
        return idxbuf[j] if idx_off is not None else sidx[i]

    def issue_gather(i, j):
        if idx_off is not None:
            for k in range(b_sz // 16):
                sl = pl.ds(k * 16, 16)
                idxbuf[j][sl] = sidx[i][sl] * 2 + idx_off
        pltpu.async_copy(y_hbm.at[gather_idx_ref(i, j)], rows[j], gsems[j])

    for i in range(IDXN):
        @pl.when(i < nblk)
        def _(i=i):
            issue_idx(i, i)

    for j in range(NBUF):
        @pl.when(j < nblk)
        def _(j=j):
            wait_idx(j)
            issue_gather(j, j)

    @pl.loop(0, nblk, step=IDXN)
    def _(g):
        for half in range(2):
            for j in range(NBUF):
                b = g + NBUF * half + j
                i = NBUF * half + j

                @pl.when(b < nblk)
                def _(b=b, i=i, j=j):
                    pltpu.make_async_copy(y_hbm.at[gather_idx_ref(i, j)],
                                          rows[j], gsems[j]).wait()
                    pltpu.async_copy(rows[j], acc.at[didx[i]], ssems[j],
                                     add=True)

            for j in range(NBUF):
                b = g + NBUF * half + j
                i = NBUF * half + j
                inext = (i + NBUF) % IDXN

                @pl.when(b < nblk)
                def _(b=b, i=i, j=j, inext=inext):
                    pltpu.make_async_copy(rows[j], acc.at[didx[i]],
                                          ssems[j]).wait()

                    @pl.when(b + NBUF < nblk)
                    def _():
                        wait_idx(inext)
                        issue_gather(inext, j)

                    @pl.when(b + IDXN < nblk)
                    def _():
                        issue_idx(b + IDXN, i)


_WB = 624
_WB_TAIL = N - NS * _WB


def _copy_sliced(s, src, dst):
    off = s * _WB
    pltpu.sync_copy(src.at[pl.ds(off, _WB)], dst.at[pl.ds(off, _WB)])

    @pl.when(s == NS - 1)
    def _():
        pltpu.sync_copy(src.at[pl.ds(NS * _WB, _WB_TAIL)],
                        dst.at[pl.ds(NS * _WB, _WB_TAIL)])


def _zero_acc(s, zeros_hbm, acc):
    _copy_sliced(s, zeros_hbm, acc)


def _core_writeback(c, s, acc, out0_hbm, out1_hbm):
    @pl.when(c == 0)
    def _():
        _copy_sliced(s, acc, out0_hbm)

    @pl.when(c == 1)
    def _():
        _copy_sliced(s, acc, out1_hbm)


def _make_deg_kernel():
    @functools.partial(
        pl.kernel,
        out_type=[jax.ShapeDtypeStruct((N, W), jnp.float32),
                  jax.ShapeDtypeStruct((N, W), jnp.float32)],
        mesh=_sc_mesh(),
        scratch_types=[pltpu.VMEM((HALF_B,), jnp.int32)] * IDXN
          + [pltpu.VMEM((HALF_B, W), jnp.float32),
             pltpu.VMEM_SHARED((N, W), jnp.float32)]
          + [pltpu.SemaphoreType.DMA] * (IDXN + NBUF),
    )
    def deg_k(dst_hbm, ones_hbm, zeros_hbm, out0_hbm, out1_hbm, *scr):
        didx = list(scr[0:IDXN])
        onesv, acc = scr[IDXN], scr[IDXN + 1]
        sems = list(scr[IDXN + 2:])
        isems, ssems = sems[:IDXN], sems[IDXN:]

        c = lax.axis_index("c")
        s = lax.axis_index("s")

        _zero_acc(s, zeros_hbm, acc)

        pltpu.sync_copy(ones_hbm, onesv)
        plsc.subcore_barrier()
        ebase = (c * NS + s) * HALF_NBLK * HALF_B
        _scatter_pass(HALF_NBLK, HALF_B, ebase, dst_hbm, onesv, didx, acc,
                      isems, ssems)
        plsc.subcore_barrier()
        _core_writeback(c, s, acc, out0_hbm, out1_hbm)

    return deg_k


def _make_layer1_kernel():
    @functools.partial(
        pl.kernel,
        out_type=[jax.ShapeDtypeStruct((N, W), jnp.float32),
                  jax.ShapeDtypeStruct((N, W), jnp.float32),
                  jax.ShapeDtypeStruct((N, W), jnp.float32),
                  jax.ShapeDtypeStruct((N, W), jnp.float32)],
        mesh=_sc_mesh(),
        scratch_types=[pltpu.VMEM((EDGE_B,), jnp.int32)] * (2 * IDXN)
          + [pltpu.VMEM((EDGE_B,), jnp.int32)] * NBUF
          + [pltpu.VMEM((HALF_B,), jnp.int32)] * (2 * IDXN)
          + [pltpu.VMEM((EDGE_B, W), jnp.float32)] * NBUF
          + [pltpu.VMEM_SHARED((N, W), jnp.float32)]
          + [pltpu.SemaphoreType.DMA] * (IDXN + 2 * NBUF),
    )
    def l1_k(x2d_hbm, u_hbm, src_hbm, dst_hbm,
             zeros_hbm, sa_hbm, sb_hbm, up0_hbm, up1_hbm, *scr):
        sidx = list(scr[0:IDXN])
        didx = list(scr[IDXN:2 * IDXN])
        idxbuf = list(scr[2 * IDXN:2 * IDXN + NBUF])
        sidxh = list(scr[2 * IDXN + NBUF:3 * IDXN + NBUF])
        didxh = list(scr[3 * IDXN + NBUF:4 * IDXN + NBUF])
        rows = list(scr[4 * IDXN + NBUF:4 * IDXN + 2 * NBUF])
        acc = scr[4 * IDXN + 2 * NBUF]
        sems = list(scr[4 * IDXN + 2 * NBUF + 1:])
        isems, gsems, ssems = sems[:IDXN], sems[IDXN:IDXN + NBUF], sems[IDXN + NBUF:]

        c = lax.axis_index("c")
        s = lax.axis_index("s")

        _zero_acc(s, zeros_hbm, acc)

        plsc.subcore_barrier()
        _gather_scatter_pass(NBLK, EDGE_B, s * NBLK * EDGE_B, src_hbm, dst_hbm,
                             x2d_hbm, sidx, didx, rows, acc,
                             isems, gsems, ssems, idx_off=c, idxbuf=idxbuf)
        plsc.subcore_barrier()
        _core_writeback(c, s, acc, sa_hbm, sb_hbm)
        plsc.subcore_barrier()

        _zero_acc(s, zeros_hbm, acc)

        plsc.subcore_barrier()
        rows40 = [r.at[pl.ds(0, HALF_B)] for r in rows]
        _gather_scatter_pass(HALF_NBLK, HALF_B,
                             (c * NS + s) * HALF_NBLK * HALF_B,
                             src_hbm, dst_hbm, u_hbm, sidxh, didxh, rows40,
                             acc, isems, gsems, ssems)
        plsc.subcore_barrier()
        _core_writeback(c, s, acc, up0_hbm, up1_hbm)

    return l1_k


def _make_layer2_kernel():
    @functools.partial(
        pl.kernel,
        out_type=[jax.ShapeDtypeStruct((N, W), jnp.float32),
                  jax.ShapeDtypeStruct((N, W), jnp.float32)],
        mesh=_sc_mesh(),
        scratch_types=[pltpu.VMEM((HALF_B,), jnp.int32)] * (2 * IDXN)
          + [pltpu.VMEM((HALF_B, W), jnp.float32)] * NBUF
          + [pltpu.VMEM_SHARED((N, W), jnp.float32)]
          + [pltpu.SemaphoreType.DMA] * (IDXN + 2 * NBUF),
    )
    def l2_k(x1_hbm, src_hbm, dst_hbm, zeros_hbm, p0_hbm, p1_hbm, *scr):
        sidx = list(scr[0:IDXN])
        didx = list(scr[IDXN:2 * IDXN])
        rows = list(scr[2 * IDXN:2 * IDXN + NBUF])
        acc = scr[2 * IDXN + NBUF]
        sems = list(scr[2 * IDXN + NBUF + 1:])
        isems, gsems, ssems = sems[:IDXN], sems[IDXN:IDXN + NBUF], sems[IDXN + NBUF:]

        c = lax.axis_index("c")
        s = lax.axis_index("s")

        _zero_acc(s, zeros_hbm, acc)

        plsc.subcore_barrier()
        _gather_scatter_pass(HALF_NBLK, HALF_B,
                             (c * NS + s) * HALF_NBLK * HALF_B,
                             src_hbm, dst_hbm, x1_hbm, sidx, didx, rows,
                             acc, isems, gsems, ssems)
        plsc.subcore_barrier()
        _core_writeback(c, s, acc, p0_hbm, p1_hbm)

    return l2_k


def _full(shape):
    return pl.BlockSpec(shape, lambda i: tuple(0 for _ in shape))


def _rows(width):
    return pl.BlockSpec((TCR, width), lambda i: (i, 0))


def _tc1(x, gcn_w, w2, deg0, deg1):
    def body(x_ref, d0_ref, d1_ref, w_ref, w2_ref, u_ref, h_ref, xw_ref):
        deg = d0_ref[:, :1] + d1_ref[:, :1]
        dinv = lax.rsqrt(deg + 1.0)
        xv = x_ref[...]
        h = jnp.dot(xv, w_ref[...], preferred_element_type=jnp.float32)
        h_ref[...] = h
        u_ref[...] = dinv * h
        xw_ref[...] = jnp.dot(xv, w2_ref[...], preferred_element_type=jnp.float32)

    return pl.pallas_call(
        body,
        grid=(N // TCR,),
        in_specs=[_rows(256), _rows(W), _rows(W),
                  _full((256, 128)), _full((256, 256))],
        out_specs=[_rows(128), _rows(128), _rows(256)],
        out_shape=[jax.ShapeDtypeStruct((N, 128), jnp.float32),
                   jax.ShapeDtypeStruct((N, 128), jnp.float32),
                   jax.ShapeDtypeStruct((N, 256), jnp.float32)],
    )(x, deg0, deg1, gcn_w, w2)


def _tc2(sa, sb, up0, up1, h, xw, deg0, deg1, w3, gcn_b, sage_b, graph_brel):
    def body(sa_ref, sb_ref, u0_ref, u1_ref, h_ref, xw_ref, d0_ref, d1_ref,
             w3_ref, gb_ref, sb2_ref, rb_ref, x1_ref, x2_ref, x3_ref):
        deg = d0_ref[:, :1] + d1_ref[:, :1]
        dinv_sl = lax.rsqrt(deg + 1.0)
        inv_sl = 1.0 / (deg + 1.0)
        sinv = 1.0 / jnp.maximum(deg, 1.0)
        nsum = jnp.concatenate([sa_ref[...], sb_ref[...]], axis=1)
        gs = u0_ref[...] + u1_ref[...]
        gcn = dinv_sl * gs + h_ref[...] * inv_sl + gb_ref[...]
        x1_ref[...] = jnp.maximum(gcn, 0.0)
        nw = jnp.dot(nsum, w3_ref[...], preferred_element_type=jnp.float32)
        xwv = xw_ref[...]
        x2_ref[...] = jnp.maximum(sinv * nw[:, :128] + xwv[:, :128] + sb2_ref[...], 0.0)
        x3_ref[...] = jnp.maximum(nw[:, 128:] + xwv[:, 128:] + rb_ref[...], 0.0)

    return pl.pallas_call(
        body,
        grid=(N // TCR,),
        in_specs=[_rows(W), _rows(W), _rows(W), _rows(W),
                  _rows(128), _rows(256), _rows(W), _rows(W),
                  _full((256, 256)),
                  _full((1, 128)), _full((1, 128)), _full((1, 128))],
        out_specs=[_rows(128), _rows(128), _rows(128)],
        out_shape=[jax.ShapeDtypeStruct((N, 128), jnp.float32),
                   jax.ShapeDtypeStruct((N, 128), jnp.float32),
                   jax.ShapeDtypeStruct((N, 128), jnp.float32)],
    )(sa, sb, up0, up1, h, xw, deg0, deg1, w3,
      gcn_b.reshape(1, 128), sage_b.reshape(1, 128), graph_brel.reshape(1, 128))


def _tc3(x1, n2p0, n2p1, x2, x3, gin_w, gin_b, out_w, out_b):
    def body(x1_ref, p0_ref, p1_ref, x2_ref, x3_ref,
             gw_ref, gb_ref, ow_ref, ob_ref, out_ref):
        z = x1_ref[...] + p0_ref[...] + p1_ref[...]
        x4 = jnp.maximum(
            jnp.dot(z, gw_ref[...], preferred_element_type=jnp.float32)
            + gb_ref[...], 0.0)
        ow = ow_ref[...]
        o = (jnp.dot(x2_ref[...], ow[:128], preferred_element_type=jnp.float32)
             + jnp.dot(x3_ref[...], ow[128:256], preferred_element_type=jnp.float32)
             + jnp.dot(x4, ow[256:], preferred_element_type=jnp.float32)
             + ob_ref[...])
        out_ref[...] = jax.nn.sigmoid(o)

    return pl.pallas_call(
        body,
        grid=(N // TCR,),
        in_specs=[_rows(128), _rows(W), _rows(W), _rows(128), _rows(128),
                  _full((128, 128)), _full((1, 128)),
                  _full((384, 256)), _full((1, 256))],
        out_specs=_rows(256),
        out_shape=jax.ShapeDtypeStruct((N, 256), jnp.float32),
    )(x1, n2p0, n2p1, x2, x3,
      gin_w, gin_b.reshape(1, 128), out_w, out_b.reshape(1, 256))


def kernel(x, edge_index, params):
    p1, p2, po = params["l1"], params["l2"], params["out"]
    src = edge_index[0]
    dst = edge_index[1]

    ones128 = jnp.ones((HALF_B, W), jnp.float32)
    zeros128 = jnp.zeros((N, W), jnp.float32)
    deg0, deg1 = _make_deg_kernel()(dst, ones128, zeros128)

    w2 = jnp.concatenate([p1["sage_wr"], p1["graph_wroot"]], axis=1)
    u, h, xw = _tc1(x, p1["gcn_w"], w2, deg0, deg1)

    x2d = x.reshape(2 * N, 128)
    sa, sb, up0, up1 = _make_layer1_kernel()(x2d, u, src, dst, zeros128)

    w3 = jnp.concatenate([p1["sage_wl"], p1["graph_wrel"]], axis=1)
    x1, x2, x3 = _tc2(sa, sb, up0, up1, h, xw, deg0, deg1, w3,
                      p1["gcn_b"], p1["sage_b"], p1["graph_brel"])

    n2p0, n2p1 = _make_layer2_kernel()(x1, src, dst, zeros128)

    return _tc3(x1, n2p0, n2p1, x2, x3,
                p2["gin_w"], p2["gin_b"], po["w"], po["b"])


# speedup vs baseline: 1.1549x; 1.0168x over previous
import functools

import jax
import jax.numpy as jnp
from jax import lax
from jax.experimental import pallas as pl
from jax.experimental.pallas import tpu as pltpu
from jax.experimental.pallas import tpu_sc as plsc

N = 10000
E = 160000
NS = 16
NC = 2
W = 128
EDGE_B = 80
NBLK = E // NS // EDGE_B
HALF_TILE = 5040
HALF_NBLK_BIG = HALF_TILE // EDGE_B
HALF_NBLK_LAST = (E // NC - (NS - 1) * HALF_TILE) // EDGE_B
NBUF = 4
TCR = 1000


def _sc_mesh():
    return plsc.VectorSubcoreMesh(core_axis_name="c", subcore_axis_name="s")


IDXN = 2 * NBUF


def _scatter_pass(nblk, b_sz, ebase, dst_hbm, onesv, didx, acc, isems, ssems):
    def issue_idx(b, i):
        off = pl.multiple_of(ebase + b * b_sz, 8)
        pltpu.async_copy(dst_hbm.at[pl.ds(off, b_sz)], didx[i], isems[i])

    def wait_idx(i):
        pltpu.make_async_copy(dst_hbm.at[pl.ds(0, b_sz)], didx[i],
                              isems[i]).wait()

    for i in range(IDXN):
        @pl.when(i < nblk)
        def _(i=i):
            issue_idx(i, i)

    @pl.loop(0, nblk, step=IDXN)
    def _(g):
        for half in range(2):
            for j in range(NBUF):
                b = g + NBUF * half + j
                i = NBUF * half + j

                @pl.when(b < nblk)
                def _(b=b, i=i, j=j):
                    wait_idx(i)
                    pltpu.async_copy(onesv, acc.at[didx[i]], ssems[j],
                                     add=True)

            for j in range(NBUF):
                b = g + NBUF * half + j
                i = NBUF * half + j

                @pl.when(b < nblk)
                def _(b=b, i=i, j=j):
                    pltpu.make_async_copy(onesv, acc.at[didx[i]],
                                          ssems[j]).wait()

                    @pl.when(b + IDXN < nblk)
                    def _():
                        issue_idx(b + IDXN, i)


def _gather_scatter_pass(nblk, b_sz, ebase, src_hbm, dst_hbm, y_hbm,
                         sidx, didx, rows, acc, isems, gsems, ssems,
                         idx_off=None, idxbuf=None):
    def issue_idx(b, i):
        off = pl.multiple_of(ebase + b * b_sz, 8)
        pltpu.async_copy(src_hbm.at[pl.ds(off, b_sz)], sidx[i], isems[i])
        pltpu.async_copy(dst_hbm.at[pl.ds(off, b_sz)], didx[i], isems[i])

    def wait_idx(i):
        pltpu.make_async_copy(src_hbm.at[pl.ds(0, b_sz)], sidx[i], isems[i]).wait()
        pltpu.make_async_copy(dst_hbm.at[pl.ds(0, b_sz)], didx[i], isems[i]).wait()

    def gather_idx_ref(i, j):
        return idxbuf[j] if idx_off is not None else sidx[i]

    def issue_gather(i, j):
        if idx_off is not None:
            for k in range(b_sz // 16):
                sl = pl.ds(k * 16, 16)
                idxbuf[j][sl] = sidx[i][sl] * 2 + idx_off
        pltpu.async_copy(y_hbm.at[gather_idx_ref(i, j)], rows[j], gsems[j])

    for i in range(IDXN):
        @pl.when(i < nblk)
        def _(i=i):
            issue_idx(i, i)

    for j in range(NBUF):
        @pl.when(j < nblk)
        def _(j=j):
            wait_idx(j)
            issue_gather(j, j)

    @pl.loop(0, nblk, step=IDXN)
    def _(g):
        for half in range(2):
            for j in range(NBUF):
                b = g + NBUF * half + j
                i = NBUF * half + j

                @pl.when(b < nblk)
                def _(b=b, i=i, j=j):
                    pltpu.make_async_copy(y_hbm.at[gather_idx_ref(i, j)],
                                          rows[j], gsems[j]).wait()
                    pltpu.async_copy(rows[j], acc.at[didx[i]], ssems[j],
                                     add=True)

            for j in range(NBUF):
                b = g + NBUF * half + j
                i = NBUF * half + j
                inext = (i + NBUF) % IDXN

                @pl.when(b < nblk)
                def _(b=b, i=i, j=j, inext=inext):
                    pltpu.make_async_copy(rows[j], acc.at[didx[i]],
                                          ssems[j]).wait()

                    @pl.when(b + NBUF < nblk)
                    def _():
                        wait_idx(inext)
                        issue_gather(inext, j)

                    @pl.when(b + IDXN < nblk)
                    def _():
                        issue_idx(b + IDXN, i)


_WB = 624
_WB_TAIL = N - NS * _WB


def _copy_sliced(s, src, dst):
    off = s * _WB
    pltpu.sync_copy(src.at[pl.ds(off, _WB)], dst.at[pl.ds(off, _WB)])

    @pl.when(s == NS - 1)
    def _():
        pltpu.sync_copy(src.at[pl.ds(NS * _WB, _WB_TAIL)],
                        dst.at[pl.ds(NS * _WB, _WB_TAIL)])


def _zero_acc(s, zeros_hbm, acc):
    _copy_sliced(s, zeros_hbm, acc)


def _core_writeback(c, s, acc, out0_hbm, out1_hbm):
    @pl.when(c == 0)
    def _():
        _copy_sliced(s, acc, out0_hbm)

    @pl.when(c == 1)
    def _():
        _copy_sliced(s, acc, out1_hbm)


def _make_deg_kernel():
    @functools.partial(
        pl.kernel,
        out_type=[jax.ShapeDtypeStruct((N, W), jnp.float32),
                  jax.ShapeDtypeStruct((N, W), jnp.float32)],
        mesh=_sc_mesh(),
        scratch_types=[pltpu.VMEM((EDGE_B,), jnp.int32)] * IDXN
          + [pltpu.VMEM((EDGE_B, W), jnp.float32),
             pltpu.VMEM_SHARED((N, W), jnp.float32)]
          + [pltpu.SemaphoreType.DMA] * (IDXN + NBUF),
    )
    def deg_k(dst_hbm, ones_hbm, zeros_hbm, out0_hbm, out1_hbm, *scr):
        didx = list(scr[0:IDXN])
        onesv, acc = scr[IDXN], scr[IDXN + 1]
        sems = list(scr[IDXN + 2:])
        isems, ssems = sems[:IDXN], sems[IDXN:]

        c = lax.axis_index("c")
        s = lax.axis_index("s")

        _zero_acc(s, zeros_hbm, acc)

        pltpu.sync_copy(ones_hbm, onesv)
        plsc.subcore_barrier()
        ebase = c * (E // NC) + s * HALF_TILE
        nblk = jnp.where(s == NS - 1, HALF_NBLK_LAST, HALF_NBLK_BIG)
        _scatter_pass(nblk, EDGE_B, ebase, dst_hbm, onesv, didx, acc,
                      isems, ssems)
        plsc.subcore_barrier()
        _core_writeback(c, s, acc, out0_hbm, out1_hbm)

    return deg_k


def _make_layer1_kernel():
    @functools.partial(
        pl.kernel,
        out_type=[jax.ShapeDtypeStruct((N, W), jnp.float32),
                  jax.ShapeDtypeStruct((N, W), jnp.float32),
                  jax.ShapeDtypeStruct((N, W), jnp.float32),
                  jax.ShapeDtypeStruct((N, W), jnp.float32)],
        mesh=_sc_mesh(),
        scratch_types=[pltpu.VMEM((EDGE_B,), jnp.int32)] * (2 * IDXN)
          + [pltpu.VMEM((EDGE_B,), jnp.int32)] * NBUF
          + [pltpu.VMEM((EDGE_B, W), jnp.float32)] * NBUF
          + [pltpu.VMEM_SHARED((N, W), jnp.float32)]
          + [pltpu.SemaphoreType.DMA] * (IDXN + 2 * NBUF),
    )
    def l1_k(x2d_hbm, u_hbm, src_hbm, dst_hbm,
             zeros_hbm, sa_hbm, sb_hbm, up0_hbm, up1_hbm, *scr):
        sidx = list(scr[0:IDXN])
        didx = list(scr[IDXN:2 * IDXN])
        idxbuf = list(scr[2 * IDXN:2 * IDXN + NBUF])
        rows = list(scr[2 * IDXN + NBUF:2 * IDXN + 2 * NBUF])
        acc = scr[2 * IDXN + 2 * NBUF]
        sems = list(scr[2 * IDXN + 2 * NBUF + 1:])
        isems, gsems, ssems = sems[:IDXN], sems[IDXN:IDXN + NBUF], sems[IDXN + NBUF:]

        c = lax.axis_index("c")
        s = lax.axis_index("s")

        _zero_acc(s, zeros_hbm, acc)

        plsc.subcore_barrier()
        _gather_scatter_pass(NBLK, EDGE_B, s * NBLK * EDGE_B, src_hbm, dst_hbm,
                             x2d_hbm, sidx, didx, rows, acc,
                             isems, gsems, ssems, idx_off=c, idxbuf=idxbuf)
        plsc.subcore_barrier()
        _core_writeback(c, s, acc, sa_hbm, sb_hbm)
        plsc.subcore_barrier()

        _zero_acc(s, zeros_hbm, acc)

        plsc.subcore_barrier()
        nblk = jnp.where(s == NS - 1, HALF_NBLK_LAST, HALF_NBLK_BIG)
        _gather_scatter_pass(nblk, EDGE_B, c * (E // NC) + s * HALF_TILE,
                             src_hbm, dst_hbm, u_hbm, sidx, didx, rows,
                             acc, isems, gsems, ssems)
        plsc.subcore_barrier()
        _core_writeback(c, s, acc, up0_hbm, up1_hbm)

    return l1_k


def _make_layer2_kernel():
    @functools.partial(
        pl.kernel,
        out_type=[jax.ShapeDtypeStruct((N, W), jnp.float32),
                  jax.ShapeDtypeStruct((N, W), jnp.float32)],
        mesh=_sc_mesh(),
        scratch_types=[pltpu.VMEM((EDGE_B,), jnp.int32)] * (2 * IDXN)
          + [pltpu.VMEM((EDGE_B, W), jnp.float32)] * NBUF
          + [pltpu.VMEM_SHARED((N, W), jnp.float32)]
          + [pltpu.SemaphoreType.DMA] * (IDXN + 2 * NBUF),
    )
    def l2_k(x1_hbm, src_hbm, dst_hbm, zeros_hbm, p0_hbm, p1_hbm, *scr):
        sidx = list(scr[0:IDXN])
        didx = list(scr[IDXN:2 * IDXN])
        rows = list(scr[2 * IDXN:2 * IDXN + NBUF])
        acc = scr[2 * IDXN + NBUF]
        sems = list(scr[2 * IDXN + NBUF + 1:])
        isems, gsems, ssems = sems[:IDXN], sems[IDXN:IDXN + NBUF], sems[IDXN + NBUF:]

        c = lax.axis_index("c")
        s = lax.axis_index("s")

        _zero_acc(s, zeros_hbm, acc)

        plsc.subcore_barrier()
        nblk = jnp.where(s == NS - 1, HALF_NBLK_LAST, HALF_NBLK_BIG)
        _gather_scatter_pass(nblk, EDGE_B, c * (E // NC) + s * HALF_TILE,
                             src_hbm, dst_hbm, x1_hbm, sidx, didx, rows,
                             acc, isems, gsems, ssems)
        plsc.subcore_barrier()
        _core_writeback(c, s, acc, p0_hbm, p1_hbm)

    return l2_k


def _full(shape):
    return pl.BlockSpec(shape, lambda i: tuple(0 for _ in shape))


def _rows(width):
    return pl.BlockSpec((TCR, width), lambda i: (i, 0))


def _tc1(x, gcn_w, w2, deg0, deg1):
    def body(x_ref, d0_ref, d1_ref, w_ref, w2_ref, u_ref, h_ref, xw_ref):
        deg = d0_ref[:, :1] + d1_ref[:, :1]
        dinv = lax.rsqrt(deg + 1.0)
        xv = x_ref[...]
        h = jnp.dot(xv, w_ref[...], preferred_element_type=jnp.float32)
        h_ref[...] = h
        u_ref[...] = dinv * h
        xw_ref[...] = jnp.dot(xv, w2_ref[...], preferred_element_type=jnp.float32)

    return pl.pallas_call(
        body,
        grid=(N // TCR,),
        in_specs=[_rows(256), _rows(W), _rows(W),
                  _full((256, 128)), _full((256, 256))],
        out_specs=[_rows(128), _rows(128), _rows(256)],
        out_shape=[jax.ShapeDtypeStruct((N, 128), jnp.float32),
                   jax.ShapeDtypeStruct((N, 128), jnp.float32),
                   jax.ShapeDtypeStruct((N, 256), jnp.float32)],
    )(x, deg0, deg1, gcn_w, w2)


def _tc2(sa, sb, up0, up1, h, xw, deg0, deg1, w3, gcn_b, sage_b, graph_brel):
    def body(sa_ref, sb_ref, u0_ref, u1_ref, h_ref, xw_ref, d0_ref, d1_ref,
             w3_ref, gb_ref, sb2_ref, rb_ref, x1_ref, x2_ref, x3_ref):
        deg = d0_ref[:, :1] + d1_ref[:, :1]
        dinv_sl = lax.rsqrt(deg + 1.0)
        inv_sl = 1.0 / (deg + 1.0)
        sinv = 1.0 / jnp.maximum(deg, 1.0)
        nsum = jnp.concatenate([sa_ref[...], sb_ref[...]], axis=1)
        gs = u0_ref[...] + u1_ref[...]
        gcn = dinv_sl * gs + h_ref[...] * inv_sl + gb_ref[...]
        x1_ref[...] = jnp.maximum(gcn, 0.0)
        nw = jnp.dot(nsum, w3_ref[...], preferred_element_type=jnp.float32)
        xwv = xw_ref[...]
        x2_ref[...] = jnp.maximum(sinv * nw[:, :128] + xwv[:, :128] + sb2_ref[...], 0.0)
        x3_ref[...] = jnp.maximum(nw[:, 128:] + xwv[:, 128:] + rb_ref[...], 0.0)

    return pl.pallas_call(
        body,
        grid=(N // TCR,),
        in_specs=[_rows(W), _rows(W), _rows(W), _rows(W),
                  _rows(128), _rows(256), _rows(W), _rows(W),
                  _full((256, 256)),
                  _full((1, 128)), _full((1, 128)), _full((1, 128))],
        out_specs=[_rows(128), _rows(128), _rows(128)],
        out_shape=[jax.ShapeDtypeStruct((N, 128), jnp.float32),
                   jax.ShapeDtypeStruct((N, 128), jnp.float32),
                   jax.ShapeDtypeStruct((N, 128), jnp.float32)],
    )(sa, sb, up0, up1, h, xw, deg0, deg1, w3,
      gcn_b.reshape(1, 128), sage_b.reshape(1, 128), graph_brel.reshape(1, 128))


def _tc3(x1, n2p0, n2p1, x2, x3, gin_w, gin_b, out_w, out_b):
    def body(x1_ref, p0_ref, p1_ref, x2_ref, x3_ref,
             gw_ref, gb_ref, ow_ref, ob_ref, out_ref):
        z = x1_ref[...] + p0_ref[...] + p1_ref[...]
        x4 = jnp.maximum(
            jnp.dot(z, gw_ref[...], preferred_element_type=jnp.float32)
            + gb_ref[...], 0.0)
        ow = ow_ref[...]
        o = (jnp.dot(x2_ref[...], ow[:128], preferred_element_type=jnp.float32)
             + jnp.dot(x3_ref[...], ow[128:256], preferred_element_type=jnp.float32)
             + jnp.dot(x4, ow[256:], preferred_element_type=jnp.float32)
             + ob_ref[...])
        out_ref[...] = jax.nn.sigmoid(o)

    return pl.pallas_call(
        body,
        grid=(N // TCR,),
        in_specs=[_rows(128), _rows(W), _rows(W), _rows(128), _rows(128),
                  _full((128, 128)), _full((1, 128)),
                  _full((384, 256)), _full((1, 256))],
        out_specs=_rows(256),
        out_shape=jax.ShapeDtypeStruct((N, 256), jnp.float32),
    )(x1, n2p0, n2p1, x2, x3,
      gin_w, gin_b.reshape(1, 128), out_w, out_b.reshape(1, 256))


def kernel(x, edge_index, params):
    p1, p2, po = params["l1"], params["l2"], params["out"]
    src = edge_index[0]
    dst = edge_index[1]

    ones128 = jnp.ones((EDGE_B, W), jnp.float32)
    zeros128 = jnp.zeros((N, W), jnp.float32)
    deg0, deg1 = _make_deg_kernel()(dst, ones128, zeros128)

    w2 = jnp.concatenate([p1["sage_wr"], p1["graph_wroot"]], axis=1)
    u, h, xw = _tc1(x, p1["gcn_w"], w2, deg0, deg1)

    x2d = x.reshape(2 * N, 128)
    sa, sb, up0, up1 = _make_layer1_kernel()(x2d, u, src, dst, zeros128)

    w3 = jnp.concatenate([p1["sage_wl"], p1["graph_wrel"]], axis=1)
    x1, x2, x3 = _tc2(sa, sb, up0, up1, h, xw, deg0, deg1, w3,
                      p1["gcn_b"], p1["sage_b"], p1["graph_brel"])

    n2p0, n2p1 = _make_layer2_kernel()(x1, src, dst, zeros128)

    return _tc3(x1, n2p0, n2p1, x2, x3,
                p2["gin_w"], p2["gin_b"], po["w"], po["b"])


# speedup vs baseline: 1.1749x; 1.0173x over previous
import functools

import jax
import jax.numpy as jnp
from jax import lax
from jax.experimental import pallas as pl
from jax.experimental.pallas import tpu as pltpu
from jax.experimental.pallas import tpu_sc as plsc

N = 10000
E = 160000
NS = 16
NC = 2
W = 128
EDGE_B = 80
NBLK = E // NS // EDGE_B
HALF_TILE = 5040
HALF_NBLK_BIG = HALF_TILE // EDGE_B
HALF_NBLK_LAST = (E // NC - (NS - 1) * HALF_TILE) // EDGE_B
NBUF = 4
TCR = 1000


def _sc_mesh():
    return plsc.VectorSubcoreMesh(core_axis_name="c", subcore_axis_name="s")


IDXN = 2 * NBUF


def _scatter_pass(nblk, b_sz, ebase, dst_hbm, onesv, didx, acc, isems, ssems):
    def issue_idx(b, i):
        off = pl.multiple_of(ebase + b * b_sz, 8)
        pltpu.async_copy(dst_hbm.at[pl.ds(off, b_sz)], didx[i], isems[i])

    def wait_idx(i):
        pltpu.make_async_copy(dst_hbm.at[pl.ds(0, b_sz)], didx[i],
                              isems[i]).wait()

    for i in range(IDXN):
        @pl.when(i < nblk)
        def _(i=i):
            issue_idx(i, i)

    @pl.loop(0, nblk, step=IDXN)
    def _(g):
        for half in range(2):
            for j in range(NBUF):
                b = g + NBUF * half + j
                i = NBUF * half + j

                @pl.when(b < nblk)
                def _(b=b, i=i, j=j):
                    wait_idx(i)
                    pltpu.async_copy(onesv, acc.at[didx[i]], ssems[j],
                                     add=True)

            for j in range(NBUF):
                b = g + NBUF * half + j
                i = NBUF * half + j

                @pl.when(b < nblk)
                def _(b=b, i=i, j=j):
                    pltpu.make_async_copy(onesv, acc.at[didx[i]],
                                          ssems[j]).wait()

                    @pl.when(b + IDXN < nblk)
                    def _():
                        issue_idx(b + IDXN, i)


def _gather_scatter_pass(nblk, b_sz, ebase, src_hbm, dst_hbm, y_hbm,
                         sidx, didx, rows, acc, isems, gsems, ssems,
                         idx_off=None, idxbuf=None):
    def issue_idx(b, i):
        off = pl.multiple_of(ebase + b * b_sz, 8)
        pltpu.async_copy(src_hbm.at[pl.ds(off, b_sz)], sidx[i], isems[i])
        pltpu.async_copy(dst_hbm.at[pl.ds(off, b_sz)], didx[i], isems[i])

    def wait_idx(i):
        pltpu.make_async_copy(src_hbm.at[pl.ds(0, b_sz)], sidx[i], isems[i]).wait()
        pltpu.make_async_copy(dst_hbm.at[pl.ds(0, b_sz)], didx[i], isems[i]).wait()

    def gather_idx_ref(i, j):
        return idxbuf[j] if idx_off is not None else sidx[i]

    def issue_gather(i, j):
        if idx_off is not None:
            for k in range(b_sz // 16):
                sl = pl.ds(k * 16, 16)
                idxbuf[j][sl] = sidx[i][sl] * 2 + idx_off
        pltpu.async_copy(y_hbm.at[gather_idx_ref(i, j)], rows[j], gsems[j])

    for i in range(IDXN):
        @pl.when(i < nblk)
        def _(i=i):
            issue_idx(i, i)

    for j in range(NBUF):
        @pl.when(j < nblk)
        def _(j=j):
            wait_idx(j)
            issue_gather(j, j)

    @pl.loop(0, nblk, step=IDXN)
    def _(g):
        for half in range(2):
            for j in range(NBUF):
                b = g + NBUF * half + j
                i = NBUF * half + j

                @pl.when(b < nblk)
                def _(b=b, i=i, j=j):
                    pltpu.make_async_copy(y_hbm.at[gather_idx_ref(i, j)],
                                          rows[j], gsems[j]).wait()
                    pltpu.async_copy(rows[j], acc.at[didx[i]], ssems[j],
                                     add=True)

            for j in range(NBUF):
                b = g + NBUF * half + j
                i = NBUF * half + j
                inext = (i + NBUF) % IDXN

                @pl.when(b < nblk)
                def _(b=b, i=i, j=j, inext=inext):
                    pltpu.make_async_copy(rows[j], acc.at[didx[i]],
                                          ssems[j]).wait()

                    @pl.when(b + NBUF < nblk)
                    def _():
                        wait_idx(inext)
                        issue_gather(inext, j)

                    @pl.when(b + IDXN < nblk)
                    def _():
                        issue_idx(b + IDXN, i)


_WB = 624
_WB_TAIL = N - NS * _WB


def _copy_sliced(s, src, dst):
    off = s * _WB
    pltpu.sync_copy(src.at[pl.ds(off, _WB)], dst.at[pl.ds(off, _WB)])

    @pl.when(s == NS - 1)
    def _():
        pltpu.sync_copy(src.at[pl.ds(NS * _WB, _WB_TAIL)],
                        dst.at[pl.ds(NS * _WB, _WB_TAIL)])


def _zero_acc(s, zeros_hbm, acc):
    _copy_sliced(s, zeros_hbm, acc)


def _core_writeback(c, s, acc, out0_hbm, out1_hbm):
    @pl.when(c == 0)
    def _():
        _copy_sliced(s, acc, out0_hbm)

    @pl.when(c == 1)
    def _():
        _copy_sliced(s, acc, out1_hbm)


def _make_deg_kernel():
    @functools.partial(
        pl.kernel,
        out_type=[jax.ShapeDtypeStruct((N, W), jnp.float32),
                  jax.ShapeDtypeStruct((N, W), jnp.float32)],
        mesh=_sc_mesh(),
        scratch_types=[pltpu.VMEM((EDGE_B,), jnp.int32)] * IDXN
          + [pltpu.VMEM((EDGE_B, W), jnp.float32),
             pltpu.VMEM_SHARED((N, W), jnp.float32)]
          + [pltpu.SemaphoreType.DMA] * (IDXN + NBUF),
    )
    def deg_k(dst_hbm, ones_hbm, zeros_hbm, out0_hbm, out1_hbm, *scr):
        didx = list(scr[0:IDXN])
        onesv, acc = scr[IDXN], scr[IDXN + 1]
        sems = list(scr[IDXN + 2:])
        isems, ssems = sems[:IDXN], sems[IDXN:]

        c = lax.axis_index("c")
        s = lax.axis_index("s")

        _zero_acc(s, zeros_hbm, acc)

        pltpu.sync_copy(ones_hbm, onesv)
        plsc.subcore_barrier()
        ebase = c * (E // NC) + s * HALF_TILE
        nblk = jnp.where(s == NS - 1, HALF_NBLK_LAST, HALF_NBLK_BIG)
        _scatter_pass(nblk, EDGE_B, ebase, dst_hbm, onesv, didx, acc,
                      isems, ssems)
        plsc.subcore_barrier()
        _core_writeback(c, s, acc, out0_hbm, out1_hbm)

    return deg_k


def _make_layer1_kernel():
    @functools.partial(
        pl.kernel,
        out_type=[jax.ShapeDtypeStruct((N, W), jnp.float32),
                  jax.ShapeDtypeStruct((N, W), jnp.float32),
                  jax.ShapeDtypeStruct((N, W), jnp.float32),
                  jax.ShapeDtypeStruct((N, W), jnp.float32)],
        mesh=_sc_mesh(),
        scratch_types=[pltpu.VMEM((EDGE_B,), jnp.int32)] * (2 * IDXN)
          + [pltpu.VMEM((EDGE_B,), jnp.int32)] * NBUF
          + [pltpu.VMEM((EDGE_B, W), jnp.float32)] * NBUF
          + [pltpu.VMEM_SHARED((N, W), jnp.float32)]
          + [pltpu.SemaphoreType.DMA] * (IDXN + 2 * NBUF),
    )
    def l1_k(x2d_hbm, u_hbm, src_hbm, dst_hbm,
             zeros_hbm, sa_hbm, sb_hbm, up0_hbm, up1_hbm, *scr):
        sidx = list(scr[0:IDXN])
        didx = list(scr[IDXN:2 * IDXN])
        idxbuf = list(scr[2 * IDXN:2 * IDXN + NBUF])
        rows = list(scr[2 * IDXN + NBUF:2 * IDXN + 2 * NBUF])
        acc = scr[2 * IDXN + 2 * NBUF]
        sems = list(scr[2 * IDXN + 2 * NBUF + 1:])
        isems, gsems, ssems = sems[:IDXN], sems[IDXN:IDXN + NBUF], sems[IDXN + NBUF:]

        c = lax.axis_index("c")
        s = lax.axis_index("s")

        _zero_acc(s, zeros_hbm, acc)

        plsc.subcore_barrier()
        _gather_scatter_pass(NBLK, EDGE_B, s * NBLK * EDGE_B, src_hbm, dst_hbm,
                             x2d_hbm, sidx, didx, rows, acc,
                             isems, gsems, ssems, idx_off=c, idxbuf=idxbuf)
        plsc.subcore_barrier()
        _core_writeback(c, s, acc, sa_hbm, sb_hbm)
        plsc.subcore_barrier()

        _zero_acc(s, zeros_hbm, acc)

        plsc.subcore_barrier()
        nblk = jnp.where(s == NS - 1, HALF_NBLK_LAST, HALF_NBLK_BIG)
        _gather_scatter_pass(nblk, EDGE_B, c * (E // NC) + s * HALF_TILE,
                             src_hbm, dst_hbm, u_hbm, sidx, didx, rows,
                             acc, isems, gsems, ssems)
        plsc.subcore_barrier()
        _core_writeback(c, s, acc, up0_hbm, up1_hbm)

    return l1_k


def _make_layer2_kernel():
    @functools.partial(
        pl.kernel,
        out_type=[jax.ShapeDtypeStruct((N, W), jnp.float32),
                  jax.ShapeDtypeStruct((N, W), jnp.float32)],
        mesh=_sc_mesh(),
        scratch_types=[pltpu.VMEM((EDGE_B,), jnp.int32)] * (2 * IDXN)
          + [pltpu.VMEM((EDGE_B, W), jnp.float32)] * NBUF
          + [pltpu.VMEM_SHARED((N, W), jnp.float32)]
          + [pltpu.SemaphoreType.DMA] * (IDXN + 2 * NBUF),
    )
    def l2_k(x1_hbm, src_hbm, dst_hbm, zeros_hbm, p0_hbm, p1_hbm, *scr):
        sidx = list(scr[0:IDXN])
        didx = list(scr[IDXN:2 * IDXN])
        rows = list(scr[2 * IDXN:2 * IDXN + NBUF])
        acc = scr[2 * IDXN + NBUF]
        sems = list(scr[2 * IDXN + NBUF + 1:])
        isems, gsems, ssems = sems[:IDXN], sems[IDXN:IDXN + NBUF], sems[IDXN + NBUF:]

        c = lax.axis_index("c")
        s = lax.axis_index("s")

        _zero_acc(s, zeros_hbm, acc)

        plsc.subcore_barrier()
        nblk = jnp.where(s == NS - 1, HALF_NBLK_LAST, HALF_NBLK_BIG)
        _gather_scatter_pass(nblk, EDGE_B, c * (E // NC) + s * HALF_TILE,
                             src_hbm, dst_hbm, x1_hbm, sidx, didx, rows,
                             acc, isems, gsems, ssems)
        plsc.subcore_barrier()
        _core_writeback(c, s, acc, p0_hbm, p1_hbm)

    return l2_k


def _full(shape):
    return pl.BlockSpec(shape, lambda i: tuple(0 for _ in shape))


def _rows(width):
    return pl.BlockSpec((TCR, width), lambda i: (i, 0))


def _tc1a(x, gcn_w, w2):
    def body(x_ref, w_ref, w2_ref, h_ref, xw_ref):
        xv = x_ref[...]
        h_ref[...] = jnp.dot(xv, w_ref[...], preferred_element_type=jnp.float32)
        xw_ref[...] = jnp.dot(xv, w2_ref[...], preferred_element_type=jnp.float32)

    return pl.pallas_call(
        body,
        grid=(N // TCR,),
        in_specs=[_rows(256), _full((256, 128)), _full((256, 256))],
        out_specs=[_rows(128), _rows(256)],
        out_shape=[jax.ShapeDtypeStruct((N, 128), jnp.float32),
                   jax.ShapeDtypeStruct((N, 256), jnp.float32)],
    )(x, gcn_w, w2)


def _tc1b(h, deg0, deg1):
    def body(h_ref, d0_ref, d1_ref, u_ref):
        deg = d0_ref[:, :1] + d1_ref[:, :1]
        u_ref[...] = lax.rsqrt(deg + 1.0) * h_ref[...]

    return pl.pallas_call(
        body,
        grid=(N // TCR,),
        in_specs=[_rows(128), _rows(W), _rows(W)],
        out_specs=_rows(128),
        out_shape=jax.ShapeDtypeStruct((N, 128), jnp.float32),
    )(h, deg0, deg1)


def _tc2(sa, sb, up0, up1, h, xw, deg0, deg1, w3, gcn_b, sage_b, graph_brel):
    def body(sa_ref, sb_ref, u0_ref, u1_ref, h_ref, xw_ref, d0_ref, d1_ref,
             w3_ref, gb_ref, sb2_ref, rb_ref, x1_ref, x2_ref, x3_ref):
        deg = d0_ref[:, :1] + d1_ref[:, :1]
        dinv_sl = lax.rsqrt(deg + 1.0)
        inv_sl = 1.0 / (deg + 1.0)
        sinv = 1.0 / jnp.maximum(deg, 1.0)
        nsum = jnp.concatenate([sa_ref[...], sb_ref[...]], axis=1)
        gs = u0_ref[...] + u1_ref[...]
        gcn = dinv_sl * gs + h_ref[...] * inv_sl + gb_ref[...]
        x1_ref[...] = jnp.maximum(gcn, 0.0)
        nw = jnp.dot(nsum, w3_ref[...], preferred_element_type=jnp.float32)
        xwv = xw_ref[...]
        x2_ref[...] = jnp.maximum(sinv * nw[:, :128] + xwv[:, :128] + sb2_ref[...], 0.0)
        x3_ref[...] = jnp.maximum(nw[:, 128:] + xwv[:, 128:] + rb_ref[...], 0.0)

    return pl.pallas_call(
        body,
        grid=(N // TCR,),
        in_specs=[_rows(W), _rows(W), _rows(W), _rows(W),
                  _rows(128), _rows(256), _rows(W), _rows(W),
                  _full((256, 256)),
                  _full((1, 128)), _full((1, 128)), _full((1, 128))],
        out_specs=[_rows(128), _rows(128), _rows(128)],
        out_shape=[jax.ShapeDtypeStruct((N, 128), jnp.float32),
                   jax.ShapeDtypeStruct((N, 128), jnp.float32),
                   jax.ShapeDtypeStruct((N, 128), jnp.float32)],
    )(sa, sb, up0, up1, h, xw, deg0, deg1, w3,
      gcn_b.reshape(1, 128), sage_b.reshape(1, 128), graph_brel.reshape(1, 128))


def _tc3a(x2, x3, out_w, out_b):
    def body(x2_ref, x3_ref, ow_ref, ob_ref, o_ref):
        ow = ow_ref[...]
        o_ref[...] = (
            jnp.dot(x2_ref[...], ow[:128], preferred_element_type=jnp.float32)
            + jnp.dot(x3_ref[...], ow[128:], preferred_element_type=jnp.float32)
            + ob_ref[...])

    return pl.pallas_call(
        body,
        grid=(N // TCR,),
        in_specs=[_rows(128), _rows(128), _full((256, 256)), _full((1, 256))],
        out_specs=_rows(256),
        out_shape=jax.ShapeDtypeStruct((N, 256), jnp.float32),
    )(x2, x3, out_w[:256], out_b.reshape(1, 256))


def _tc3b(x1, n2p0, n2p1, oacc, gin_w, gin_b, out_w):
    def body(x1_ref, p0_ref, p1_ref, o_ref, gw_ref, gb_ref, ow_ref, out_ref):
        z = x1_ref[...] + p0_ref[...] + p1_ref[...]
        x4 = jnp.maximum(
            jnp.dot(z, gw_ref[...], preferred_element_type=jnp.float32)
            + gb_ref[...], 0.0)
        o = o_ref[...] + jnp.dot(x4, ow_ref[...],
                                 preferred_element_type=jnp.float32)
        out_ref[...] = jax.nn.sigmoid(o)

    return pl.pallas_call(
        body,
        grid=(N // TCR,),
        in_specs=[_rows(128), _rows(W), _rows(W), _rows(256),
                  _full((128, 128)), _full((1, 128)), _full((128, 256))],
        out_specs=_rows(256),
        out_shape=jax.ShapeDtypeStruct((N, 256), jnp.float32),
    )(x1, n2p0, n2p1, oacc, gin_w, gin_b.reshape(1, 128), out_w[256:])


def kernel(x, edge_index, params):
    p1, p2, po = params["l1"], params["l2"], params["out"]
    src = edge_index[0]
    dst = edge_index[1]

    ones128 = jnp.ones((EDGE_B, W), jnp.float32)
    zeros128 = jnp.zeros((N, W), jnp.float32)
    w2 = jnp.concatenate([p1["sage_wr"], p1["graph_wroot"]], axis=1)
    h, xw = _tc1a(x, p1["gcn_w"], w2)

    deg0, deg1 = _make_deg_kernel()(dst, ones128, zeros128)
    u = _tc1b(h, deg0, deg1)

    x2d = x.reshape(2 * N, 128)
    sa, sb, up0, up1 = _make_layer1_kernel()(x2d, u, src, dst, zeros128)

    w3 = jnp.concatenate([p1["sage_wl"], p1["graph_wrel"]], axis=1)
    x1, x2, x3 = _tc2(sa, sb, up0, up1, h, xw, deg0, deg1, w3,
                      p1["gcn_b"], p1["sage_b"], p1["graph_brel"])

    oacc = _tc3a(x2, x3, po["w"], po["b"])
    n2p0, n2p1 = _make_layer2_kernel()(x1, src, dst, zeros128)

    return _tc3b(x1, n2p0, n2p1, oacc, p2["gin_w"], p2["gin_b"], po["w"])
